# baseline (speedup 1.0000x reference)
import functools

import jax
import jax.numpy as jnp
from jax.experimental import pallas as pl
from jax.experimental.pallas import tpu as pltpu
from jax.experimental.pallas import tpu_sc as plsc

_B, _N, _K = 16, 256, 20
_KP = 24
_EPS = 1e-5
_NEG = -3.0e38

_NW = 32
_PPW = (_B * _N) // _NW


def _pairwise(xb):
    inner = jnp.dot(xb, xb.T, preferred_element_type=jnp.float32)
    xx = jnp.sum(xb * xb, axis=1)
    return 2.0 * inner - xx[:, None] - xx[None, :]


def _features_kernel(x_ref, W1_ref, g1_ref, b1_ref, W2_ref, g2_ref, b2_ref,
                     W3_ref, g3_ref, b3_ref, y3_ref, idxt_ref,
                     idx_ref, cnt_ref):
    N, K = _N, _K
    iota_m = jax.lax.broadcasted_iota(jnp.int32, (N, N), 0)

    def phase1(b, carry):
        xb = x_ref[pl.ds(b, 1)].reshape(N, 8)
        P = _pairwise(xb)

        def step(i, cur):
            vmax = jnp.max(cur, axis=0, keepdims=True)
            ii = jnp.min(jnp.where(cur >= vmax, iota_m, N), axis=0,
                         keepdims=True)
            idx_ref[pl.ds(b * 32 + i, 1), :] = ii

            @pl.when(i < _KP - K)
            def _():
                idx_ref[pl.ds(b * 32 + K + i, 1), :] = ii

            return jnp.where(iota_m == ii, _NEG, cur)

        cur = jax.lax.fori_loop(0, K, step, P)
        maskf = (cur <= -1.0e37).astype(jnp.float32)
        cnt_ref[pl.ds(b * N, N), :] = jnp.dot(
            maskf, jnp.ones((N, 8), jnp.float32),
            preferred_element_type=jnp.float32)
        blk = idx_ref[pl.ds(b * 32, _KP), :]
        idxt_ref[pl.ds(b * N, N), :] = jnp.transpose(blk) + b * N
        return carry

    jax.lax.fori_loop(0, _B, phase1, 0)

    cnt = cnt_ref[:, 0:1]

    scale = 1.0 / float(_B * N * K)
    tdims = (((0,), (0,)), ((), ()))

    def bn_relu(z, g_ref, beta_ref):
        s1 = jax.lax.dot_general(cnt, z, tdims,
                                 preferred_element_type=jnp.float32)
        s2 = jax.lax.dot_general(cnt, z * z, tdims,
                                 preferred_element_type=jnp.float32)
        mean = s1 * scale
        var = s2 * scale - mean * mean
        y = (z - mean) * jax.lax.rsqrt(var + _EPS)
        y = y * g_ref[:] + beta_ref[:]
        return jnp.maximum(y, 0.0)

    X = x_ref[:].reshape(_B * N, 8)
    z1 = jnp.dot(X, W1_ref[:].T, preferred_element_type=jnp.float32)
    y1 = bn_relu(z1, g1_ref, b1_ref)
    z2 = jnp.dot(y1, W2_ref[:].T, preferred_element_type=jnp.float32)
    y2 = bn_relu(z2, g2_ref, b2_ref)
    z3 = jnp.dot(y2, W3_ref[:].T, preferred_element_type=jnp.float32)
    y3_ref[:] = bn_relu(z3, g3_ref, b3_ref)


_G = 4
_NG = _PPW // _G


@functools.partial(
    pl.kernel,
    out_type=jax.ShapeDtypeStruct((_B * _N, 256), jnp.float32),
    mesh=plsc.VectorSubcoreMesh(core_axis_name="c", subcore_axis_name="s"),
    scratch_types=[
        pltpu.VMEM((_PPW * _KP,), jnp.int32),
        pltpu.VMEM((_G * _KP, 256), jnp.float32),
        pltpu.VMEM((_G * _KP, 256), jnp.float32),
        pltpu.VMEM((_PPW, 256), jnp.float32),
        pltpu.SemaphoreType.DMA,
        pltpu.SemaphoreType.DMA,
    ],
)
def _pool_sc(y3_hbm, idx_hbm, out_hbm, idx_v, rows0, rows1, out_v,
             sem0, sem1):
    wid = jax.lax.axis_index("s") * 2 + jax.lax.axis_index("c")
    base = wid * _PPW
    pltpu.sync_copy(idx_hbm.at[pl.ds(base * _KP, _PPW * _KP)], idx_v)

    def dsc(g, buf, sem):
        return pltpu.make_async_copy(
            y3_hbm.at[idx_v.at[pl.ds(g * _G * _KP, _G * _KP)]], buf, sem)

    def compute(g, buf):
        for pt in range(_G):
            p = g * _G + pt
            for c in range(16):
                acc = buf[pt * _KP, pl.ds(c * 16, 16)]
                for r in range(1, _KP):
                    acc = jnp.maximum(acc, buf[pt * _KP + r,
                                               pl.ds(c * 16, 16)])
                out_v[p, pl.ds(c * 16, 16)] = acc

    dsc(0, rows0, sem0).start()

    def body(h, carry):
        g0 = 2 * h
        g1 = 2 * h + 1
        dsc(g1, rows1, sem1).start()
        dsc(g0, rows0, sem0).wait()
        compute(g0, rows0)

        @pl.when(g1 + 1 < _NG)
        def _():
            dsc(g1 + 1, rows0, sem0).start()

        dsc(g1, rows1, sem1).wait()
        compute(g1, rows1)
        return carry

    jax.lax.fori_loop(0, _NG // 2, body, 0)
    pltpu.sync_copy(out_v, out_hbm.at[pl.ds(base, _PPW)])


def _head_kernel(pool_ref, fc1w_ref, fc1b_ref, fc2w_ref, fc2b_ref, out_ref):
    N = _N
    fc1w = fc1w_ref[:]
    fc1b = fc1b_ref[:]
    fc2w = fc2w_ref[:]
    fc2b = fc2b_ref[:]

    def body(b, carry):
        pool_b = pool_ref[pl.ds(b * N, N), :].astype(jnp.float32)
        h = jax.lax.dot_general(pool_b, fc1w, (((0,), (1,)), ((), ())),
                                preferred_element_type=jnp.float32)
        h = jnp.maximum(h + fc1b, 0.0)
        o = jax.lax.dot_general(h, fc2w, (((1,), (1,)), ((), ())),
                                preferred_element_type=jnp.float32)
        out_ref[pl.ds(b, 1)] = (o + fc2b).reshape(1, N, 40)
        return carry

    jax.lax.fori_loop(0, _B, body, 0)


def kernel(x, W1, g1, b1, W2, g2, b2, W3, g3, b3, fc1_w, fc1_b, fc2_w, fc2_b):
    xp = jnp.pad(x, ((0, 0), (0, 0), (0, 5)))
    W1p = jnp.pad(W1, ((0, 0), (0, 5)))
    y3, idxt = pl.pallas_call(
        _features_kernel,
        out_shape=(jax.ShapeDtypeStruct((_B * _N, 256), jnp.float32),
                   jax.ShapeDtypeStruct((_B * _N, _KP), jnp.int32)),
        scratch_shapes=[pltpu.VMEM((_B * 32, _N), jnp.int32),
                        pltpu.VMEM((_B * _N, 8), jnp.float32)],
    )(xp,
      W1p, g1.reshape(1, -1), b1.reshape(1, -1),
      W2, g2.reshape(1, -1), b2.reshape(1, -1),
      W3, g3.reshape(1, -1), b3.reshape(1, -1))
    pool = _pool_sc(y3, idxt.reshape(-1))
    return pl.pallas_call(
        _head_kernel,
        out_shape=jax.ShapeDtypeStruct((_B, _N, 40), jnp.float32),
    )(pool, fc1_w, fc1_b.reshape(1, -1), fc2_w, fc2_b.reshape(1, -1))


# speedup vs baseline: 1.0753x; 1.0753x over previous
import functools

import jax
import jax.numpy as jnp
from jax.experimental import pallas as pl
from jax.experimental.pallas import tpu as pltpu
from jax.experimental.pallas import tpu_sc as plsc

_B, _N, _K = 16, 256, 20
_KP = 24
_EPS = 1e-5
_NEG = -3.0e38
_SPLIT = 0

_NW = 32
_SCPTS = (_B - _SPLIT) * _N
_PPW = _SCPTS // _NW
_G = 2
_NG = _PPW // _G


def _pairwise(xb):
    inner = jnp.dot(xb, xb.T, preferred_element_type=jnp.float32)
    xx = jnp.sum(xb * xb, axis=1)
    return 2.0 * inner - xx[:, None] - xx[None, :]


def _features_kernel(x_ref, W1_ref, g1_ref, b1_ref, W2_ref, g2_ref, b2_ref,
                     W3_ref, g3_ref, b3_ref, y3_ref, idxrows_ref, idxt_ref,
                     cnt_ref):
    N, K = _N, _K
    iota_m = jax.lax.broadcasted_iota(jnp.int32, (N, N), 0)

    def phase1(b, carry):
        xb = x_ref[pl.ds(b, 1)].reshape(N, 8)
        P = _pairwise(xb)

        def step(i, cur):
            vmax = jnp.max(cur, axis=0, keepdims=True)
            ii = jnp.min(jnp.where(cur >= vmax, iota_m, N), axis=0,
                         keepdims=True)
            idxrows_ref[pl.ds(b * 32 + i, 1), :] = ii

            @pl.when(i < _KP - K)
            def _():
                idxrows_ref[pl.ds(b * 32 + K + i, 1), :] = ii

            return jnp.where(iota_m == ii, _NEG, cur)

        cur = jax.lax.fori_loop(0, K, step, P)
        maskf = (cur <= -1.0e37).astype(jnp.float32)
        cnt_ref[pl.ds(b * N, N), :] = jnp.dot(
            maskf, jnp.ones((N, 8), jnp.float32),
            preferred_element_type=jnp.float32)
        blk = idxrows_ref[pl.ds(b * 32, _KP), :]
        idxt_ref[pl.ds(b * N, N), :] = jnp.transpose(blk) + b * N
        return carry

    jax.lax.fori_loop(0, _B, phase1, 0)

    cnt = cnt_ref[:, 0:1]

    scale = 1.0 / float(_B * N * K)
    tdims = (((0,), (0,)), ((), ()))

    def bn_relu(z, g_ref, beta_ref):
        s1 = jax.lax.dot_general(cnt, z, tdims,
                                 preferred_element_type=jnp.float32)
        s2 = jax.lax.dot_general(cnt, z * z, tdims,
                                 preferred_element_type=jnp.float32)
        mean = s1 * scale
        var = s2 * scale - mean * mean
        y = (z - mean) * jax.lax.rsqrt(var + _EPS)
        y = y * g_ref[:] + beta_ref[:]
        return jnp.maximum(y, 0.0)

    X = x_ref[:].reshape(_B * N, 8)
    z1 = jnp.dot(X, W1_ref[:].T, preferred_element_type=jnp.float32)
    y1 = bn_relu(z1, g1_ref, b1_ref)
    z2 = jnp.dot(y1, W2_ref[:].T, preferred_element_type=jnp.float32)
    y2 = bn_relu(z2, g2_ref, b2_ref)
    z3 = jnp.dot(y2, W3_ref[:].T, preferred_element_type=jnp.float32)
    y3_ref[:] = bn_relu(z3, g3_ref, b3_ref)


@functools.partial(
    pl.kernel,
    out_type=jax.ShapeDtypeStruct((_B * _N, 256), jnp.float32),
    mesh=plsc.VectorSubcoreMesh(core_axis_name="c", subcore_axis_name="s"),
    scratch_types=[
        pltpu.VMEM((_PPW * _KP,), jnp.int32),
        pltpu.VMEM((_G * _KP, 256), jnp.float32),
        pltpu.VMEM((_G * _KP, 256), jnp.float32),
        pltpu.VMEM((_PPW, 256), jnp.float32),
        pltpu.SemaphoreType.DMA,
        pltpu.SemaphoreType.DMA,
    ],
)
def _pool_sc(y3_hbm, idx_hbm, out_hbm, idx_v, rows0, rows1, out_v,
             sem0, sem1):
    wid = jax.lax.axis_index("s") * 2 + jax.lax.axis_index("c")
    base = _SPLIT * _N + wid * _PPW
    pltpu.sync_copy(idx_hbm.at[pl.ds(base * _KP, _PPW * _KP)], idx_v)

    def dsc(g, buf, sem):
        return pltpu.make_async_copy(
            y3_hbm.at[idx_v.at[pl.ds(g * _G * _KP, _G * _KP)]], buf, sem)

    def compute(g, buf):
        for pt in range(_G):
            p = g * _G + pt
            for c in range(16):
                acc = buf[pt * _KP, pl.ds(c * 16, 16)]
                for r in range(1, _KP):
                    acc = jnp.maximum(acc, buf[pt * _KP + r,
                                               pl.ds(c * 16, 16)])
                out_v[p, pl.ds(c * 16, 16)] = acc

    dsc(0, rows0, sem0).start()

    def body(h, carry):
        g0 = 2 * h
        g1 = 2 * h + 1
        dsc(g1, rows1, sem1).start()
        dsc(g0, rows0, sem0).wait()
        compute(g0, rows0)

        @pl.when(g1 + 1 < _NG)
        def _():
            dsc(g1 + 1, rows0, sem0).start()

        dsc(g1, rows1, sem1).wait()
        compute(g1, rows1)
        return carry

    jax.lax.fori_loop(0, _NG // 2, body, 0)
    pltpu.sync_copy(out_v, out_hbm.at[pl.ds(base, _PPW)])


def _tc_pool_kernel(y3_ref, idxrows_ref, fc1w_ref, fc1b_ref, fc2w_ref,
                    fc2b_ref, out_ref, ycat_ref):
    N, K = _N, _K
    iota_m = jax.lax.broadcasted_iota(jnp.int32, (N, N), 0)
    tdims = (((0,), (0,)), ((), ()))

    y3 = y3_ref[:]
    y_hi = y3.astype(jnp.bfloat16)
    y_lo = (y3 - y_hi.astype(jnp.float32)).astype(jnp.bfloat16)
    ycat_ref[:] = jnp.concatenate([y_hi, y_lo], axis=1)

    fc1w = fc1w_ref[:]
    fc1b = fc1b_ref[:]
    fc2w = fc2w_ref[:]
    fc2b = fc2b_ref[:]

    def pool_b(b, carry):
        ycat_b = ycat_ref[pl.ds(b * N, N), :]

        def step(i, acc):
            ii_row = idxrows_ref[pl.ds(b * 32 + i, 1), :]
            oh = iota_m == ii_row
            sel2 = jax.lax.dot_general(
                oh.astype(jnp.bfloat16), ycat_b, tdims,
                preferred_element_type=jnp.float32)
            sel = sel2[:, 0:256] + sel2[:, 256:512]
            return jnp.maximum(acc, sel)

        acc = jax.lax.fori_loop(0, K, step, jnp.zeros((N, 256), jnp.float32))
        h = jax.lax.dot_general(acc, fc1w, (((0,), (1,)), ((), ())),
                                preferred_element_type=jnp.float32)
        h = jnp.maximum(h + fc1b, 0.0)
        o = jax.lax.dot_general(h, fc2w, (((1,), (1,)), ((), ())),
                                preferred_element_type=jnp.float32)
        out_ref[pl.ds(b, 1)] = (o + fc2b).reshape(1, N, 40)
        return carry

    jax.lax.fori_loop(0, _SPLIT, pool_b, 0)


def _head_kernel(pool_ref, fc1w_ref, fc1b_ref, fc2w_ref, fc2b_ref, out_ref):
    N = _N
    fc1w = fc1w_ref[:]
    fc1b = fc1b_ref[:]
    fc2w = fc2w_ref[:]
    fc2b = fc2b_ref[:]

    def body(b, carry):
        pool_b = pool_ref[pl.ds(b * N, N), :]
        h = jax.lax.dot_general(pool_b, fc1w, (((0,), (1,)), ((), ())),
                                preferred_element_type=jnp.float32)
        h = jnp.maximum(h + fc1b, 0.0)
        o = jax.lax.dot_general(h, fc2w, (((1,), (1,)), ((), ())),
                                preferred_element_type=jnp.float32)
        out_ref[pl.ds(b, 1)] = (o + fc2b).reshape(1, N, 40)
        return carry

    jax.lax.fori_loop(_SPLIT, _B, body, 0)


def kernel(x, W1, g1, b1, W2, g2, b2, W3, g3, b3, fc1_w, fc1_b, fc2_w, fc2_b):
    xp = jnp.pad(x, ((0, 0), (0, 0), (0, 5)))
    W1p = jnp.pad(W1, ((0, 0), (0, 5)))
    y3, idxrows, idxt = pl.pallas_call(
        _features_kernel,
        out_shape=(jax.ShapeDtypeStruct((_B * _N, 256), jnp.float32),
                   jax.ShapeDtypeStruct((_B * 32, _N), jnp.int32),
                   jax.ShapeDtypeStruct((_B * _N, _KP), jnp.int32)),
        scratch_shapes=[pltpu.VMEM((_B * _N, 8), jnp.float32)],
    )(xp,
      W1p, g1.reshape(1, -1), b1.reshape(1, -1),
      W2, g2.reshape(1, -1), b2.reshape(1, -1),
      W3, g3.reshape(1, -1), b3.reshape(1, -1))
    fc1b2 = fc1_b.reshape(1, -1)
    fc2b2 = fc2_b.reshape(1, -1)
    pool = _pool_sc(y3, idxt.reshape(-1))
    out_tc = pl.pallas_call(
        _tc_pool_kernel,
        out_shape=jax.ShapeDtypeStruct((_B, _N, 40), jnp.float32),
        scratch_shapes=[pltpu.VMEM((_B * _N, 512), jnp.bfloat16)],
    )(y3, idxrows, fc1_w, fc1b2, fc2_w, fc2b2)
    out_sc = pl.pallas_call(
        _head_kernel,
        out_shape=jax.ShapeDtypeStruct((_B, _N, 40), jnp.float32),
    )(pool, fc1_w, fc1b2, fc2_w, fc2b2)
    return jnp.concatenate([out_tc[:_SPLIT], out_sc[_SPLIT:]], axis=0)
